# baseline (speedup 1.0000x reference)
import numpy as np

import jax
import jax.numpy as jnp
from jax import lax
from jax.experimental import pallas as pl
from jax.experimental.pallas import tpu as pltpu
from jax.experimental.pallas import tpu_sc as plsc

_H = 64
_W = 64
_A = 9
_N = _A * _H * _W
_R, _C = 288, 128
_NUM_FG = 256 // 3
_NUM_BG = 256 * 2 // 3
_NGT = 32


def _build_anchor_consts():
    ws, hs = [], []
    for s in [4.0, 8.0, 16.0]:
        for r in [0.5, 1.0, 2.0]:
            ws.append(s * np.sqrt(r))
            hs.append(s / np.sqrt(r))
    ws = np.asarray(ws, np.float32)
    hs = np.asarray(hs, np.float32)
    ys, xs = np.meshgrid(np.arange(_H, dtype=np.float32),
                         np.arange(_W, dtype=np.float32), indexing='ij')
    x = (xs[None, :, :] - ws[:, None, None] / np.float32(2.0)).astype(np.float32)
    y = (ys[None, :, :] - hs[:, None, None] / np.float32(2.0)).astype(np.float32)
    w = np.broadcast_to(ws[:, None, None], x.shape).astype(np.float32)
    h = np.broadcast_to(hs[:, None, None], x.shape).astype(np.float32)
    a = np.stack([x, y, w, h], axis=-1).reshape(-1, 4)
    ax, ay, aw, ah = a[:, 0], a[:, 1], a[:, 2], a[:, 3]
    ax2 = (ax + aw - np.float32(1.0)).astype(np.float32)
    ay2 = (ay + ah - np.float32(1.0)).astype(np.float32)
    keep = ((ax >= 0) & (ay >= 0)
            & (ax2 <= np.float32(_H - 1.0)) & (ay2 <= np.float32(_H - 1.0)))
    area = (aw * ah).astype(np.float32)
    cons = np.stack([ax, ay, ax2, ay2, aw, ah, area,
                     keep.astype(np.float32)], axis=0)
    return cons.reshape(8, _R, _C)


def _threefry2x32(k0, k1, x0, x1):
    def rotl(x, d):
        return ((x << np.uint32(d)) | (x >> np.uint32(32 - d))).astype(np.uint32)

    ks = (np.uint32(k0), np.uint32(k1),
          np.uint32(0x1BD11BDA) ^ np.uint32(k0) ^ np.uint32(k1))
    x0 = (x0 + ks[0]).astype(np.uint32)
    x1 = (x1 + ks[1]).astype(np.uint32)
    rot1, rot2 = (13, 15, 26, 6), (17, 29, 16, 24)

    def rounds(x0, x1, rots):
        for r in rots:
            x0 = (x0 + x1).astype(np.uint32)
            x1 = rotl(x1, r) ^ x0
        return x0, x1

    for i, rots in enumerate((rot1, rot2, rot1, rot2, rot1)):
        x0, x1 = rounds(x0, x1, rots)
        x0 = (x0 + ks[(i + 1) % 3]).astype(np.uint32)
        x1 = (x1 + ks[(i + 2) % 3] + np.uint32(i + 1)).astype(np.uint32)
    return x0, x1


def _build_rank_consts():
    b1, b2 = _threefry2x32(np.uint32(0), np.uint32(42),
                           np.zeros(2, np.uint32), np.arange(2, dtype=np.uint32))

    def uniform(k0, k1):
        o1, o2 = _threefry2x32(k0, k1, np.zeros(_N, np.uint32),
                               np.arange(_N, dtype=np.uint32))
        bits = o1 ^ o2
        fb = (bits >> np.uint32(9)) | np.uint32(0x3F800000)
        return fb.view(np.float32) - np.float32(1.0)

    ub = uniform(b1[0], b2[0])
    uf = uniform(b1[1], b2[1])

    permb = np.argsort(ub, kind='stable').astype(np.int32)
    permf = np.argsort(uf, kind='stable').astype(np.int32)
    return permb, permf


_CONS = _build_anchor_consts()
_PERMB, _PERMF = _build_rank_consts()
_NV = _N // 16


_B = 8
_NCH = _R // _B


def _anchor_kernel(gt_ref, cons_ref, lab_ref, t_ref,
                   best_ref, gxs_ref, gys_ref, gws_ref, ghs_ref,
                   acc_ref, idx_ref, cm_ref, ci_ref):
    base = (lax.broadcasted_iota(jnp.int32, (_B, _C), 0) * _C
            + lax.broadcasted_iota(jnp.int32, (_B, _C), 1))

    for g in range(_NGT):
        acc_ref[g] = jnp.full((_B, _C), -jnp.inf, jnp.float32)
        idx_ref[g] = jnp.zeros((_B, _C), jnp.int32)

    def chunk_work(rc):
        s = rc * _B
        ax = cons_ref[0, pl.ds(s, _B), :]
        ay = cons_ref[1, pl.ds(s, _B), :]
        ax2 = cons_ref[2, pl.ds(s, _B), :]
        ay2 = cons_ref[3, pl.ds(s, _B), :]
        area = cons_ref[6, pl.ds(s, _B), :]
        keep = cons_ref[7, pl.ds(s, _B), :] != 0.0
        lin = base + rc * (_B * _C)
        chains = [None] * 4
        for g in range(_NGT):
            gx = gt_ref[g, 0]
            gy = gt_ref[g, 1]
            gw = gt_ref[g, 2]
            gh = gt_ref[g, 3]
            gx2 = gx + gw - 1.0
            gy2 = gy + gh - 1.0
            ag = gw * gh
            iw = jnp.maximum(0.0,
                             jnp.minimum(ax2, gx2) - jnp.maximum(ax, gx) + 1.0)
            ih = jnp.maximum(0.0,
                             jnp.minimum(ay2, gy2) - jnp.maximum(ay, gy) + 1.0)
            inter = iw * ih
            iou = inter / (area + ag - inter)
            ov = jnp.where(iou == 0.0, jnp.float32(1e-10), iou)
            mk = jnp.where(keep, ov, jnp.float32(-1.0))
            c = g // 8
            if chains[c] is None:
                chains[c] = (ov,
                             jnp.full((_B, _C), gx), jnp.full((_B, _C), gy),
                             jnp.full((_B, _C), gw), jnp.full((_B, _C), gh))
            else:
                best, gxs, gys, gws, ghs = chains[c]
                b = ov > best
                chains[c] = (jnp.where(b, ov, best),
                             jnp.where(b, gx, gxs), jnp.where(b, gy, gys),
                             jnp.where(b, gw, gws), jnp.where(b, gh, ghs))
            a = acc_ref[g]
            b2 = mk > a
            acc_ref[g] = jnp.where(b2, mk, a)
            idx_ref[g] = jnp.where(b2, lin, idx_ref[g])
        def merge(s0, s1):
            b = s1[0] > s0[0]
            return tuple(jnp.where(b, x1, x0) for x0, x1 in zip(s0, s1))

        best, gxs, gys, gws, ghs = merge(merge(chains[0], chains[1]),
                                         merge(chains[2], chains[3]))
        best_ref[pl.ds(s, _B), :] = best
        gxs_ref[pl.ds(s, _B), :] = gxs
        gys_ref[pl.ds(s, _B), :] = gys
        gws_ref[pl.ds(s, _B), :] = gws
        ghs_ref[pl.ds(s, _B), :] = ghs

    def chunk_body(rc, carry):
        chunk_work(rc)
        return carry

    lax.fori_loop(0, _NCH, chunk_body, 0)

    for g in range(_NGT):
        a = acc_ref[g]
        v = jnp.max(a, axis=0, keepdims=True)
        ii = jnp.min(jnp.where(a == v, idx_ref[g], jnp.int32(_N)),
                     axis=0, keepdims=True)
        cm_ref[pl.ds(g, 1), :] = v
        ci_ref[pl.ds(g, 1), :] = ii
    m32 = cm_ref[...]
    rm = jnp.max(m32, axis=1, keepdims=True)
    ci = jnp.min(jnp.where(m32 == rm, ci_ref[...], jnp.int32(_N)),
                 axis=1, keepdims=True)
    ci_ref[:, 0:1] = ci

    lin_full = (lax.broadcasted_iota(jnp.int32, (_R, _C), 0) * _C
                + lax.broadcasted_iota(jnp.int32, (_R, _C), 1))
    isgt = jnp.zeros((_R, _C), jnp.bool_)
    for g in range(_NGT):
        isgt = isgt | (lin_full == ci_ref[g, 0])

    keep = cons_ref[7] != 0.0
    ax = cons_ref[0]
    ay = cons_ref[1]
    aw = cons_ref[4]
    ah = cons_ref[5]
    mo = best_ref[...]
    gxs = gxs_ref[...]
    gys = gys_ref[...]
    gws = gws_ref[...]
    ghs = ghs_ref[...]

    lab = jnp.full((_R, _C), -1, jnp.int32)
    lab = jnp.where(mo >= 0.7, 1, lab)
    lab = jnp.where(mo <= 0.3, 0, lab)
    lab = jnp.where(isgt, 1, lab)
    lab = jnp.where(keep, lab, -1)

    lab_ref[...] = lab
    t_ref[0] = (gxs - ax) / aw
    t_ref[1] = (gys - ay) / ah
    t_ref[2] = jnp.log(gws / aw)
    t_ref[3] = jnp.log(ghs / ah)


def _sc_subsample(lab_hbm, permb_hbm, permf_hbm, out_hbm, lab_v, perm_v):
    wid = (lax.axis_index("s") * plsc.get_sparse_core_info().num_cores
           + lax.axis_index("c"))

    @pl.when(wid == 0)
    def _():
        pltpu.sync_copy(lab_hbm, lab_v)

        def do_pass(perm_hbm, which, quota):
            pltpu.sync_copy(perm_hbm, perm_v)

            def body(j, carry):
                iv = perm_v[j]
                vals = plsc.load_gather(lab_v, [iv])
                m = vals == which
                mi = m.astype(jnp.int32)
                cs = plsc.cumsum(mi) + carry
                kill = m & (cs > quota)
                newv = jnp.where(kill, jnp.int32(-1), vals)
                plsc.store_scatter(lab_v, [iv], newv)
                return carry + jnp.sum(mi)

            lax.fori_loop(0, _NV, body, jnp.int32(0))

        do_pass(permb_hbm, 0, _NUM_BG)
        do_pass(permf_hbm, 1, _NUM_FG)
        pltpu.sync_copy(lab_v, out_hbm)


def _mask_kernel(lab_ref, t_ref, out_ref):
    fg = lab_ref[...] == 1
    for k in range(4):
        out_ref[k] = jnp.where(fg, t_ref[k], 0.0)


def kernel(cls_scores, gt_boxes, image_info):
    del cls_scores, image_info
    gt = gt_boxes[0].astype(jnp.float32)
    lab, t = pl.pallas_call(
        _anchor_kernel,
        out_shape=(jax.ShapeDtypeStruct((_R, _C), jnp.int32),
                   jax.ShapeDtypeStruct((4, _R, _C), jnp.float32)),
        in_specs=[pl.BlockSpec(memory_space=pltpu.SMEM),
                  pl.BlockSpec(memory_space=pltpu.VMEM)],
        out_specs=(pl.BlockSpec(memory_space=pltpu.VMEM),
                   pl.BlockSpec(memory_space=pltpu.VMEM)),
        scratch_shapes=[pltpu.VMEM((_R, _C), jnp.float32)] * 5
                       + [pltpu.VMEM((_NGT, _B, _C), jnp.float32),
                          pltpu.VMEM((_NGT, _B, _C), jnp.int32),
                          pltpu.VMEM((_NGT, _C), jnp.float32),
                          pltpu.VMEM((_NGT, _C), jnp.int32)],
    )(gt, jnp.asarray(_CONS))
    lab_fin = pl.kernel(
        _sc_subsample,
        out_type=jax.ShapeDtypeStruct((_N,), jnp.int32),
        mesh=plsc.VectorSubcoreMesh(core_axis_name="c", subcore_axis_name="s"),
        compiler_params=pltpu.CompilerParams(use_tc_tiling_on_sc=False,
                                             needs_layout_passes=False),
        scratch_types=[pltpu.VMEM((_N,), jnp.int32),
                       pltpu.VMEM((_NV, 16), jnp.int32)],
    )(lab.reshape(_N),
      jnp.asarray(_PERMB.reshape(_NV, 16)),
      jnp.asarray(_PERMF.reshape(_NV, 16)))
    t_masked = pl.pallas_call(
        _mask_kernel,
        out_shape=jax.ShapeDtypeStruct((4, _R, _C), jnp.float32),
    )(lab_fin.reshape(_R, _C), t)
    labels_op = lab_fin.reshape(1, _A, _H, _W)
    target_op = jnp.moveaxis(t_masked, 0, -1).reshape(1, _A, _H, _W, 4)
    return labels_op, target_op


# speedup vs baseline: 6.7011x; 6.7011x over previous
import numpy as np

import jax
import jax.numpy as jnp
from jax import lax
from jax.experimental import pallas as pl
from jax.experimental.pallas import tpu as pltpu

_H = 64
_W = 64
_A = 9
_N = _A * _H * _W
_R, _C = 288, 128
_NUM_FG = 256 // 3
_NUM_BG = 256 * 2 // 3
_NGT = 32


def _build_anchor_consts():
    ws, hs = [], []
    for s in [4.0, 8.0, 16.0]:
        for r in [0.5, 1.0, 2.0]:
            ws.append(s * np.sqrt(r))
            hs.append(s / np.sqrt(r))
    ws = np.asarray(ws, np.float32)
    hs = np.asarray(hs, np.float32)
    ys, xs = np.meshgrid(np.arange(_H, dtype=np.float32),
                         np.arange(_W, dtype=np.float32), indexing='ij')
    x = (xs[None, :, :] - ws[:, None, None] / np.float32(2.0)).astype(np.float32)
    y = (ys[None, :, :] - hs[:, None, None] / np.float32(2.0)).astype(np.float32)
    w = np.broadcast_to(ws[:, None, None], x.shape).astype(np.float32)
    h = np.broadcast_to(hs[:, None, None], x.shape).astype(np.float32)
    a = np.stack([x, y, w, h], axis=-1).reshape(-1, 4)
    ax, ay, aw, ah = a[:, 0], a[:, 1], a[:, 2], a[:, 3]
    ax2 = (ax + aw - np.float32(1.0)).astype(np.float32)
    ay2 = (ay + ah - np.float32(1.0)).astype(np.float32)
    keep = ((ax >= 0) & (ay >= 0)
            & (ax2 <= np.float32(_H - 1.0)) & (ay2 <= np.float32(_H - 1.0)))
    area = (aw * ah).astype(np.float32)
    cons = np.stack([ax, ay, ax2, ay2, aw, ah, area,
                     keep.astype(np.float32)], axis=0)
    return cons.reshape(8, _R, _C)


def _threefry2x32(k0, k1, x0, x1):
    def rotl(x, d):
        return ((x << np.uint32(d)) | (x >> np.uint32(32 - d))).astype(np.uint32)

    ks = (np.uint32(k0), np.uint32(k1),
          np.uint32(0x1BD11BDA) ^ np.uint32(k0) ^ np.uint32(k1))
    x0 = (x0 + ks[0]).astype(np.uint32)
    x1 = (x1 + ks[1]).astype(np.uint32)
    rot1, rot2 = (13, 15, 26, 6), (17, 29, 16, 24)

    def rounds(x0, x1, rots):
        for r in rots:
            x0 = (x0 + x1).astype(np.uint32)
            x1 = rotl(x1, r) ^ x0
        return x0, x1

    for i, rots in enumerate((rot1, rot2, rot1, rot2, rot1)):
        x0, x1 = rounds(x0, x1, rots)
        x0 = (x0 + ks[(i + 1) % 3]).astype(np.uint32)
        x1 = (x1 + ks[(i + 2) % 3] + np.uint32(i + 1)).astype(np.uint32)
    return x0, x1


def _build_rank_consts():
    b1, b2 = _threefry2x32(np.uint32(0), np.uint32(42),
                           np.zeros(2, np.uint32), np.arange(2, dtype=np.uint32))

    def uniform(k0, k1):
        o1, o2 = _threefry2x32(k0, k1, np.zeros(_N, np.uint32),
                               np.arange(_N, dtype=np.uint32))
        bits = o1 ^ o2
        fb = (bits >> np.uint32(9)) | np.uint32(0x3F800000)
        return fb.view(np.float32) - np.float32(1.0)

    ub = uniform(b1[0], b2[0])
    uf = uniform(b1[1], b2[1])

    def ranks(u):
        perm = np.argsort(u, kind='stable')
        pos = np.empty(_N, np.int32)
        pos[perm] = np.arange(_N, dtype=np.int32)
        return pos

    return np.stack([ranks(ub).reshape(_R, _C), ranks(uf).reshape(_R, _C)], 0)


_CONS = _build_anchor_consts()
_POS = _build_rank_consts()


_B = 8
_NCH = _R // _B


def _anchor_kernel(gt_ref, cons_ref, pos_ref, lab_ref, t_ref,
                   best_ref, gxs_ref, gys_ref, gws_ref, ghs_ref,
                   acc_ref, idx_ref, cm_ref, ci_ref):
    base = (lax.broadcasted_iota(jnp.int32, (_B, _C), 0) * _C
            + lax.broadcasted_iota(jnp.int32, (_B, _C), 1))

    for g in range(_NGT):
        acc_ref[g] = jnp.full((_B, _C), -jnp.inf, jnp.float32)
        idx_ref[g] = jnp.zeros((_B, _C), jnp.int32)

    def chunk_work(rc):
        s = rc * _B
        ax = cons_ref[0, pl.ds(s, _B), :]
        ay = cons_ref[1, pl.ds(s, _B), :]
        ax2 = cons_ref[2, pl.ds(s, _B), :]
        ay2 = cons_ref[3, pl.ds(s, _B), :]
        area = cons_ref[6, pl.ds(s, _B), :]
        keep = cons_ref[7, pl.ds(s, _B), :] != 0.0
        lin = base + rc * (_B * _C)
        best = gxs = gys = gws = ghs = None
        for g in range(_NGT):
            gx = gt_ref[g, 0]
            gy = gt_ref[g, 1]
            gw = gt_ref[g, 2]
            gh = gt_ref[g, 3]
            gx2 = gx + gw - 1.0
            gy2 = gy + gh - 1.0
            ag = gw * gh
            iw = jnp.maximum(0.0,
                             jnp.minimum(ax2, gx2) - jnp.maximum(ax, gx) + 1.0)
            ih = jnp.maximum(0.0,
                             jnp.minimum(ay2, gy2) - jnp.maximum(ay, gy) + 1.0)
            inter = iw * ih
            iou = inter / (area + ag - inter)
            ov = jnp.where(iou == 0.0, jnp.float32(1e-10), iou)
            mk = jnp.where(keep, ov, jnp.float32(-1.0))
            if g == 0:
                best = ov
                gxs = jnp.full((_B, _C), gx)
                gys = jnp.full((_B, _C), gy)
                gws = jnp.full((_B, _C), gw)
                ghs = jnp.full((_B, _C), gh)
            else:
                b = ov > best
                best = jnp.where(b, ov, best)
                gxs = jnp.where(b, gx, gxs)
                gys = jnp.where(b, gy, gys)
                gws = jnp.where(b, gw, gws)
                ghs = jnp.where(b, gh, ghs)
            a = acc_ref[g]
            b2 = mk > a
            acc_ref[g] = jnp.where(b2, mk, a)
            idx_ref[g] = jnp.where(b2, lin, idx_ref[g])
        best_ref[pl.ds(s, _B), :] = best
        gxs_ref[pl.ds(s, _B), :] = gxs
        gys_ref[pl.ds(s, _B), :] = gys
        gws_ref[pl.ds(s, _B), :] = gws
        ghs_ref[pl.ds(s, _B), :] = ghs

    def chunk_body(rc, carry):
        chunk_work(rc)
        return carry

    lax.fori_loop(0, _NCH, chunk_body, 0)

    for g in range(_NGT):
        a = acc_ref[g]
        v = jnp.max(a, axis=0, keepdims=True)
        ii = jnp.min(jnp.where(a == v, idx_ref[g], jnp.int32(_N)),
                     axis=0, keepdims=True)
        cm_ref[pl.ds(g, 1), :] = v
        ci_ref[pl.ds(g, 1), :] = ii
    m32 = cm_ref[...]
    rm = jnp.max(m32, axis=1, keepdims=True)
    ci = jnp.min(jnp.where(m32 == rm, ci_ref[...], jnp.int32(_N)),
                 axis=1, keepdims=True)
    ci_ref[:, 0:1] = ci

    lin_full = (lax.broadcasted_iota(jnp.int32, (_R, _C), 0) * _C
                + lax.broadcasted_iota(jnp.int32, (_R, _C), 1))
    isgt = jnp.zeros((_R, _C), jnp.bool_)
    for g in range(_NGT):
        isgt = isgt | (lin_full == ci_ref[g, 0])

    keep = cons_ref[7] != 0.0
    ax = cons_ref[0]
    ay = cons_ref[1]
    aw = cons_ref[4]
    ah = cons_ref[5]
    mo = best_ref[...]
    gxs = gxs_ref[...]
    gys = gys_ref[...]
    gws = gws_ref[...]
    ghs = ghs_ref[...]

    lab = jnp.full((_R, _C), -1, jnp.int32)
    lab = jnp.where(mo >= 0.7, 1, lab)
    lab = jnp.where(mo <= 0.3, 0, lab)
    lab = jnp.where(isgt, 1, lab)
    lab = jnp.where(keep, lab, -1)

    m_bg = lab == 0
    m_fg = lab == 1
    pos_b = pos_ref[0]
    pos_f = pos_ref[1]
    q_bg = jnp.minimum(jnp.sum(m_bg.astype(jnp.int32)), jnp.int32(_NUM_BG))
    q_fg = jnp.minimum(jnp.sum(m_fg.astype(jnp.int32)), jnp.int32(_NUM_FG))

    def bstep(_, st):
        lob, hib, lof, hif = st
        midb = (lob + hib) // 2
        midf = (lof + hif) // 2
        cb = jnp.sum((m_bg & (pos_b <= midb)).astype(jnp.int32))
        cf = jnp.sum((m_fg & (pos_f <= midf)).astype(jnp.int32))
        tb = cb >= q_bg
        tf = cf >= q_fg
        return (jnp.where(tb, lob, midb), jnp.where(tb, midb, hib),
                jnp.where(tf, lof, midf), jnp.where(tf, midf, hif))

    _, thr_b, _, thr_f = lax.fori_loop(
        0, 16, bstep, (jnp.int32(-1), jnp.int32(_N - 1),
                       jnp.int32(-1), jnp.int32(_N - 1)))
    lab = jnp.where(m_bg & (pos_b > thr_b), -1, lab)
    lab = jnp.where(m_fg & (pos_f > thr_f), -1, lab)

    fg = lab == 1
    lab_ref[...] = lab
    t_ref[0] = jnp.where(fg, (gxs - ax) / aw, 0.0)
    t_ref[1] = jnp.where(fg, (gys - ay) / ah, 0.0)
    t_ref[2] = jnp.where(fg, jnp.log(gws / aw), 0.0)
    t_ref[3] = jnp.where(fg, jnp.log(ghs / ah), 0.0)


def kernel(cls_scores, gt_boxes, image_info):
    del cls_scores, image_info
    gt = gt_boxes[0].astype(jnp.float32)
    lab, t = pl.pallas_call(
        _anchor_kernel,
        out_shape=(jax.ShapeDtypeStruct((_R, _C), jnp.int32),
                   jax.ShapeDtypeStruct((4, _R, _C), jnp.float32)),
        in_specs=[pl.BlockSpec(memory_space=pltpu.SMEM),
                  pl.BlockSpec(memory_space=pltpu.VMEM),
                  pl.BlockSpec(memory_space=pltpu.VMEM)],
        out_specs=(pl.BlockSpec(memory_space=pltpu.VMEM),
                   pl.BlockSpec(memory_space=pltpu.VMEM)),
        scratch_shapes=[pltpu.VMEM((_R, _C), jnp.float32)] * 5
                       + [pltpu.VMEM((_NGT, _B, _C), jnp.float32),
                          pltpu.VMEM((_NGT, _B, _C), jnp.int32),
                          pltpu.VMEM((_NGT, _C), jnp.float32),
                          pltpu.VMEM((_NGT, _C), jnp.int32)],
    )(gt, jnp.asarray(_CONS), jnp.asarray(_POS))
    labels_op = lab.reshape(1, _A, _H, _W)
    target_op = jnp.moveaxis(t, 0, -1).reshape(1, _A, _H, _W, 4)
    return labels_op, target_op
